# baseline (speedup 1.0000x reference)
import functools

import jax
import jax.numpy as jnp
from jax import lax
from jax.experimental import pallas as pl
from jax.experimental.pallas import tpu as pltpu
from jax.experimental.pallas import tpu_sc as plsc

N = 10000
E = 320000
D = 128
H = 32
C = 40

NC = 2
NS = 16
NW = NC * NS
EB = 128
EPW = E // NW
CHUNKS = 80
EPW_PAD = CHUNKS * EB
NP = 10112
RPT = NP // NS

BN_ROWS = 2000
GRID = N // BN_ROWS


def _sc_agg_body(u_hbm, src_hbm, dst_hbm, zeros_hbm, out_hbm,
                 sidx_v, didx_v, rows0_v, rows1_v, stage_v, acc_sh,
                 sem0, sem1):
    c = lax.axis_index("c")
    s = lax.axis_index("s")
    wid = c * NS + s

    pltpu.sync_copy(zeros_hbm.at[pl.ds(s * RPT, RPT)], stage_v)
    pltpu.sync_copy(stage_v, acc_sh.at[pl.ds(s * RPT, RPT)])

    pltpu.sync_copy(src_hbm.at[wid], sidx_v)
    pltpu.sync_copy(dst_hbm.at[wid], didx_v)
    plsc.subcore_barrier()

    pltpu.async_copy(u_hbm.at[sidx_v.at[0]], rows0_v, sem0)
    pltpu.async_copy(u_hbm.at[sidx_v.at[1]], rows1_v, sem1)

    def body(g, carry):
        j0 = 2 * g
        pltpu.make_async_copy(u_hbm.at[sidx_v.at[j0]], rows0_v, sem0).wait()
        pltpu.sync_copy(rows0_v, acc_sh.at[didx_v.at[j0]], add=True)

        @pl.when(g + 1 < CHUNKS // 2)
        def _():
            pltpu.async_copy(u_hbm.at[sidx_v.at[j0 + 2]], rows0_v, sem0)

        pltpu.make_async_copy(u_hbm.at[sidx_v.at[j0 + 1]], rows1_v,
                              sem1).wait()
        pltpu.sync_copy(rows1_v, acc_sh.at[didx_v.at[j0 + 1]], add=True)

        @pl.when(g + 1 < CHUNKS // 2)
        def _():
            pltpu.async_copy(u_hbm.at[sidx_v.at[j0 + 3]], rows1_v, sem1)

        return carry

    lax.fori_loop(0, CHUNKS // 2, body, 0)
    plsc.subcore_barrier()

    pltpu.sync_copy(acc_sh.at[pl.ds(s * RPT, RPT)], stage_v)
    pltpu.sync_copy(stage_v, out_hbm.at[c, pl.ds(s * RPT, RPT)])


_sc_agg = functools.partial(
    pl.kernel,
    out_type=jax.ShapeDtypeStruct((NC, NP, H), jnp.float32),
    mesh=plsc.VectorSubcoreMesh(core_axis_name="c", subcore_axis_name="s",
                                num_cores=NC, num_subcores=NS),
    scratch_types=[
        pltpu.VMEM((CHUNKS, EB), jnp.int32),
        pltpu.VMEM((CHUNKS, EB), jnp.int32),
        pltpu.VMEM((EB, H), jnp.float32),
        pltpu.VMEM((EB, H), jnp.float32),
        pltpu.VMEM((RPT, H), jnp.float32),
        pltpu.VMEM_SHARED((NP, H), jnp.float32),
        pltpu.SemaphoreType.DMA,
        pltpu.SemaphoreType.DMA,
    ],
    compiler_params=pltpu.CompilerParams(use_tc_tiling_on_sc=False),
)(_sc_agg_body)


def _mm_body(x_ref, w_ref, o_ref):
    o_ref[...] = jnp.dot(x_ref[...], w_ref[...],
                         preferred_element_type=jnp.float32, precision=jax.lax.Precision.HIGHEST)


def _conv_post_body(u_ref, a0_ref, a1_ref, ba_ref, wb_ref, bb_ref,
                    h_ref, s_ref, s2_ref):
    z = jnp.maximum(u_ref[...] + a0_ref[...] + a1_ref[...] + ba_ref[...], 0.0)
    h = jnp.dot(z, wb_ref[...], preferred_element_type=jnp.float32, precision=jax.lax.Precision.HIGHEST) + bb_ref[...]
    h_ref[...] = h

    @pl.when(pl.program_id(0) == 0)
    def _():
        s_ref[...] = jnp.zeros_like(s_ref)
        s2_ref[...] = jnp.zeros_like(s2_ref)

    hr = h.reshape(BN_ROWS // 8, 8, H)
    s_ref[...] += jnp.sum(hr, axis=0)
    s2_ref[...] += jnp.sum(hr * hr, axis=0)


def _bn_mm_body(h_ref, s_ref, s2_ref, g_ref, be_ref, w_ref, o_ref):
    sm = jnp.sum(s_ref[...], axis=0, keepdims=True)
    sq = jnp.sum(s2_ref[...], axis=0, keepdims=True)
    m = sm / N
    var = sq / N - m * m
    scale = g_ref[...] * jax.lax.rsqrt(var + 1e-5)
    shift = be_ref[...] - m * scale
    hn = h_ref[...] * scale + shift
    o_ref[...] = jnp.dot(hn, w_ref[...], preferred_element_type=jnp.float32, precision=jax.lax.Precision.HIGHEST)


def _head_body(h_ref, s_ref, s2_ref, g_ref, be_ref, wf1_ref, bf1_ref,
               wf2_ref, bf2_ref, o_ref):
    sm = jnp.sum(s_ref[...], axis=0, keepdims=True)
    sq = jnp.sum(s2_ref[...], axis=0, keepdims=True)
    m = sm / N
    var = sq / N - m * m
    scale = g_ref[...] * jax.lax.rsqrt(var + 1e-5)
    shift = be_ref[...] - m * scale
    hn = h_ref[...] * scale + shift
    f = jnp.maximum(
        jnp.dot(hn, wf1_ref[...], preferred_element_type=jnp.float32, precision=jax.lax.Precision.HIGHEST)
        + bf1_ref[...], 0.0)
    o_ref[...] = jnp.dot(f, wf2_ref[...],
                         preferred_element_type=jnp.float32, precision=jax.lax.Precision.HIGHEST) + bf2_ref[...]


def _row_spec(width):
    return pl.BlockSpec((BN_ROWS, width), lambda i: (i, 0))


def _full_spec(shape):
    return pl.BlockSpec(shape, lambda i: tuple(0 for _ in shape))


def _mm(x, w, in_width, out_width):
    return pl.pallas_call(
        _mm_body,
        grid=(GRID,),
        in_specs=[_row_spec(in_width), _full_spec(w.shape)],
        out_specs=_row_spec(out_width),
        out_shape=jax.ShapeDtypeStruct((N, out_width), jnp.float32),
    )(x, w)


def _conv_post(u, agg, ba, wb, bb):
    return pl.pallas_call(
        _conv_post_body,
        grid=(GRID,),
        in_specs=[_row_spec(H), _row_spec(H), _row_spec(H),
                  _full_spec((1, H)), _full_spec((H, H)), _full_spec((1, H))],
        out_specs=[_row_spec(H), _full_spec((8, H)), _full_spec((8, H))],
        out_shape=[jax.ShapeDtypeStruct((N, H), jnp.float32),
                   jax.ShapeDtypeStruct((8, H), jnp.float32),
                   jax.ShapeDtypeStruct((8, H), jnp.float32)],
    )(u, agg[0], agg[1], ba.reshape(1, H), wb, bb.reshape(1, H))


def _bn_mm(h, s, s2, g, be, w):
    return pl.pallas_call(
        _bn_mm_body,
        grid=(GRID,),
        in_specs=[_row_spec(H), _full_spec((8, H)), _full_spec((8, H)),
                  _full_spec((1, H)), _full_spec((1, H)), _full_spec((H, H))],
        out_specs=_row_spec(H),
        out_shape=jax.ShapeDtypeStruct((N, H), jnp.float32),
    )(h, s, s2, g.reshape(1, H), be.reshape(1, H), w)


def _head(h, s, s2, g, be, wf1, bf1, wf2, bf2):
    return pl.pallas_call(
        _head_body,
        grid=(GRID,),
        in_specs=[_row_spec(H), _full_spec((8, H)), _full_spec((8, H)),
                  _full_spec((1, H)), _full_spec((1, H)),
                  _full_spec((H, H)), _full_spec((1, H)),
                  _full_spec((H, C)), _full_spec((1, C))],
        out_specs=_row_spec(C),
        out_shape=jax.ShapeDtypeStruct((N, C), jnp.float32),
    )(h, s, s2, g.reshape(1, H), be.reshape(1, H),
      wf1, bf1.reshape(1, H), wf2, bf2.reshape(1, C))


def kernel(x, edge_index, W1a, b1a, W1b, b1b, g1, be1,
           W2a, b2a, W2b, b2b, g2, be2, Wf1, bf1, Wf2, bf2):
    src = edge_index[0]
    dst = edge_index[1]
    ppw = EPW_PAD - EPW
    src_p = jnp.concatenate(
        [src.reshape(NW, EPW), jnp.zeros((NW, ppw), jnp.int32)],
        axis=1).reshape(NW, CHUNKS, EB)
    dum = N + jnp.arange(ppw, dtype=jnp.int32) % (NP - N)
    dst_p = jnp.concatenate(
        [dst.reshape(NW, EPW), jnp.broadcast_to(dum, (NW, ppw))],
        axis=1).reshape(NW, CHUNKS, EB)
    zeros = jnp.zeros((NP, H), jnp.float32)

    u = _mm(x, W1a, D, H)
    agg = _sc_agg(u, src_p, dst_p, zeros)
    h1, s1, s1sq = _conv_post(u, agg, b1a, W1b, b1b)
    v2 = _bn_mm(h1, s1, s1sq, g1, be1, W2a)
    agg2 = _sc_agg(v2, src_p, dst_p, zeros)
    h2, s2, s2sq = _conv_post(v2, agg2, b2a, W2b, b2b)
    return _head(h2, s2, s2sq, g2, be2, Wf1, bf1, Wf2, bf2)


# speedup vs baseline: 1.0007x; 1.0007x over previous
import functools

import jax
import jax.numpy as jnp
from jax import lax
from jax.experimental import pallas as pl
from jax.experimental.pallas import tpu as pltpu
from jax.experimental.pallas import tpu_sc as plsc

N = 10000
E = 320000
D = 128
H = 32
C = 40

NC = 2
NS = 16
NW = NC * NS
EB = 128
EPW = E // NW
CHUNKS = 80
EPW_PAD = CHUNKS * EB
NP = 10112
RPT = NP // NS

BN_ROWS = 2000
GRID = N // BN_ROWS


def _sc_agg_body(u_hbm, src_hbm, dst_hbm, zeros_hbm, out_hbm,
                 sidx_v, didx_v, rows0_v, rows1_v, stage_v, acc_sh,
                 sem0, sem1):
    c = lax.axis_index("c")
    s = lax.axis_index("s")
    wid = c * NS + s

    pltpu.sync_copy(zeros_hbm.at[pl.ds(s * RPT, RPT)], stage_v)
    pltpu.sync_copy(stage_v, acc_sh.at[pl.ds(s * RPT, RPT)])

    pltpu.sync_copy(src_hbm.at[wid], sidx_v)
    pltpu.sync_copy(dst_hbm.at[wid], didx_v)
    plsc.subcore_barrier()

    pltpu.async_copy(u_hbm.at[sidx_v.at[0]], rows0_v, sem0)
    pltpu.async_copy(u_hbm.at[sidx_v.at[1]], rows1_v, sem1)

    def body(g, carry):
        j0 = 2 * g
        pltpu.make_async_copy(u_hbm.at[sidx_v.at[j0]], rows0_v, sem0).wait()
        pltpu.sync_copy(rows0_v, acc_sh.at[didx_v.at[j0]], add=True)

        @pl.when(g + 1 < CHUNKS // 2)
        def _():
            pltpu.async_copy(u_hbm.at[sidx_v.at[j0 + 2]], rows0_v, sem0)

        pltpu.make_async_copy(u_hbm.at[sidx_v.at[j0 + 1]], rows1_v,
                              sem1).wait()
        pltpu.sync_copy(rows1_v, acc_sh.at[didx_v.at[j0 + 1]], add=True)

        @pl.when(g + 1 < CHUNKS // 2)
        def _():
            pltpu.async_copy(u_hbm.at[sidx_v.at[j0 + 3]], rows1_v, sem1)

        return carry

    lax.fori_loop(0, CHUNKS // 2, body, 0)
    plsc.subcore_barrier()

    pltpu.sync_copy(acc_sh.at[pl.ds(s * RPT, RPT)], stage_v)
    pltpu.sync_copy(stage_v, out_hbm.at[c, pl.ds(s * RPT, RPT)])


_sc_agg = functools.partial(
    pl.kernel,
    out_type=jax.ShapeDtypeStruct((NC, NP, H), jnp.float32),
    mesh=plsc.VectorSubcoreMesh(core_axis_name="c", subcore_axis_name="s",
                                num_cores=NC, num_subcores=NS),
    scratch_types=[
        pltpu.VMEM((CHUNKS, EB), jnp.int32),
        pltpu.VMEM((CHUNKS, EB), jnp.int32),
        pltpu.VMEM((EB, H), jnp.float32),
        pltpu.VMEM((EB, H), jnp.float32),
        pltpu.VMEM((RPT, H), jnp.float32),
        pltpu.VMEM_SHARED((NP, H), jnp.float32),
        pltpu.SemaphoreType.DMA,
        pltpu.SemaphoreType.DMA,
    ],
    compiler_params=pltpu.CompilerParams(use_tc_tiling_on_sc=False),
)(_sc_agg_body)


def _mm_body(x_ref, w_ref, o_ref):
    o_ref[...] = jnp.dot(x_ref[...], w_ref[...],
                         preferred_element_type=jnp.float32, precision=jax.lax.Precision.HIGHEST)


def _conv_post_body(u_ref, a0_ref, a1_ref, ba_ref, wb_ref, bb_ref,
                    h_ref, s_ref, s2_ref):
    z = jnp.maximum(u_ref[...] + a0_ref[...] + a1_ref[...] + ba_ref[...], 0.0)
    h = jnp.dot(z, wb_ref[...], preferred_element_type=jnp.float32, precision=jax.lax.Precision.HIGHEST) + bb_ref[...]
    h_ref[...] = h

    @pl.when(pl.program_id(0) == 0)
    def _():
        s_ref[...] = jnp.zeros_like(s_ref)
        s2_ref[...] = jnp.zeros_like(s2_ref)

    hr = h.reshape(BN_ROWS // 8, 8, H)
    s_ref[...] += jnp.sum(hr, axis=0)
    s2_ref[...] += jnp.sum(hr * hr, axis=0)


def _bn_mm_body(h_ref, s_ref, s2_ref, g_ref, be_ref, w_ref, o_ref):
    sm = jnp.sum(s_ref[...], axis=0, keepdims=True)
    sq = jnp.sum(s2_ref[...], axis=0, keepdims=True)
    m = sm / N
    var = sq / N - m * m
    scale = g_ref[...] * jax.lax.rsqrt(var + 1e-5)
    shift = be_ref[...] - m * scale
    hn = h_ref[...] * scale + shift
    o_ref[...] = jnp.dot(hn, w_ref[...], preferred_element_type=jnp.float32, precision=jax.lax.Precision.HIGHEST)


def _head_body(h_ref, s_ref, s2_ref, g_ref, be_ref, wf1_ref, bf1_ref,
               wf2_ref, bf2_ref, o_ref):
    sm = jnp.sum(s_ref[...], axis=0, keepdims=True)
    sq = jnp.sum(s2_ref[...], axis=0, keepdims=True)
    m = sm / N
    var = sq / N - m * m
    scale = g_ref[...] * jax.lax.rsqrt(var + 1e-5)
    shift = be_ref[...] - m * scale
    hn = h_ref[...] * scale + shift
    f = jnp.maximum(
        jnp.dot(hn, wf1_ref[...], preferred_element_type=jnp.float32, precision=jax.lax.Precision.HIGHEST)
        + bf1_ref[...], 0.0)
    o_ref[...] = jnp.dot(f, wf2_ref[...],
                         preferred_element_type=jnp.float32, precision=jax.lax.Precision.HIGHEST) + bf2_ref[...]


def _row_spec(width):
    return pl.BlockSpec((BN_ROWS, width), lambda i: (i, 0))


def _full_spec(shape):
    return pl.BlockSpec(shape, lambda i: tuple(0 for _ in shape))


def _mm(x, w, in_width, out_width):
    return pl.pallas_call(
        _mm_body,
        grid=(GRID,),
        in_specs=[_row_spec(in_width), _full_spec(w.shape)],
        out_specs=_row_spec(out_width),
        out_shape=jax.ShapeDtypeStruct((N, out_width), jnp.float32),
    )(x, w)


def _conv_post(u, agg, ba, wb, bb):
    return pl.pallas_call(
        _conv_post_body,
        grid=(GRID,),
        in_specs=[_row_spec(H), _row_spec(H), _row_spec(H),
                  _full_spec((1, H)), _full_spec((H, H)), _full_spec((1, H))],
        out_specs=[_row_spec(H), _full_spec((8, H)), _full_spec((8, H))],
        out_shape=[jax.ShapeDtypeStruct((N, H), jnp.float32),
                   jax.ShapeDtypeStruct((8, H), jnp.float32),
                   jax.ShapeDtypeStruct((8, H), jnp.float32)],
    )(u, agg[0], agg[1], ba.reshape(1, H), wb, bb.reshape(1, H))


def _bn_mm(h, s, s2, g, be, w):
    return pl.pallas_call(
        _bn_mm_body,
        grid=(GRID,),
        in_specs=[_row_spec(H), _full_spec((8, H)), _full_spec((8, H)),
                  _full_spec((1, H)), _full_spec((1, H)), _full_spec((H, H))],
        out_specs=_row_spec(H),
        out_shape=jax.ShapeDtypeStruct((N, H), jnp.float32),
    )(h, s, s2, g.reshape(1, H), be.reshape(1, H), w)


def _head(h, s, s2, g, be, wf1, bf1, wf2, bf2):
    return pl.pallas_call(
        _head_body,
        grid=(GRID,),
        in_specs=[_row_spec(H), _full_spec((8, H)), _full_spec((8, H)),
                  _full_spec((1, H)), _full_spec((1, H)),
                  _full_spec((H, H)), _full_spec((1, H)),
                  _full_spec((H, C)), _full_spec((1, C))],
        out_specs=_row_spec(C),
        out_shape=jax.ShapeDtypeStruct((N, C), jnp.float32),
    )(h, s, s2, g.reshape(1, H), be.reshape(1, H),
      wf1, bf1.reshape(1, H), wf2, bf2.reshape(1, C))


def kernel(x, edge_index, W1a, b1a, W1b, b1b, g1, be1,
           W2a, b2a, W2b, b2b, g2, be2, Wf1, bf1, Wf2, bf2):
    src = edge_index[0]
    dst = edge_index[1]
    ppw = EPW_PAD - EPW
    src_p = jnp.concatenate(
        [src.reshape(NW, EPW), jnp.zeros((NW, ppw), jnp.int32)],
        axis=1).reshape(NW, CHUNKS, EB)
    w = jnp.arange(NW, dtype=jnp.int32)[:, None]
    dum = N + (w % NS) * 7 + jnp.arange(ppw, dtype=jnp.int32)[None, :] % 7
    dst_p = jnp.concatenate(
        [dst.reshape(NW, EPW), dum], axis=1).reshape(NW, CHUNKS, EB)
    zeros = jnp.zeros((NP, H), jnp.float32)

    u = _mm(x, W1a, D, H)
    agg = _sc_agg(u, src_p, dst_p, zeros)
    h1, s1, s1sq = _conv_post(u, agg, b1a, W1b, b1b)
    v2 = _bn_mm(h1, s1, s1sq, g1, be1, W2a)
    agg2 = _sc_agg(v2, src_p, dst_p, zeros)
    h2, s2, s2sq = _conv_post(v2, agg2, b2a, W2b, b2b)
    return _head(h2, s2, s2sq, g2, be2, Wf1, bf1, Wf2, bf2)


# speedup vs baseline: 1.5881x; 1.5871x over previous
import functools

import jax
import jax.numpy as jnp
from jax import lax
from jax.experimental import pallas as pl
from jax.experimental.pallas import tpu as pltpu
from jax.experimental.pallas import tpu_sc as plsc

N = 10000
E = 320000
D = 128
H = 32
C = 40

NC = 2
NS = 16
NW = NC * NS
EB = 128
EPW = E // NW
CHUNKS = 80
EPW_PAD = CHUNKS * EB
NP = 10112
RPT = NP // NS

BN_ROWS = 2000
GRID = N // BN_ROWS


def _sc_agg_body(u_hbm, src_hbm, dst_hbm, zeros_hbm, out_hbm,
                 sidx_v, didx_v, rows0_v, rows1_v, stage_v, acc_sh,
                 sem0, sem1):
    c = lax.axis_index("c")
    s = lax.axis_index("s")
    wid = c * NS + s

    pltpu.sync_copy(zeros_hbm.at[pl.ds(s * RPT, RPT)], stage_v)
    pltpu.sync_copy(stage_v, acc_sh.at[pl.ds(s * RPT, RPT)])

    pltpu.sync_copy(src_hbm.at[wid], sidx_v)
    pltpu.sync_copy(dst_hbm.at[wid], didx_v)
    plsc.subcore_barrier()

    pltpu.async_copy(u_hbm.at[sidx_v.at[0]], rows0_v, sem0)
    pltpu.async_copy(u_hbm.at[sidx_v.at[1]], rows1_v, sem1)

    def body(g, carry):
        j0 = 2 * g
        pltpu.make_async_copy(u_hbm.at[sidx_v.at[j0]], rows0_v, sem0).wait()
        pltpu.sync_copy(rows0_v, acc_sh.at[didx_v.at[j0]], add=True)

        @pl.when(g + 1 < CHUNKS // 2)
        def _():
            pltpu.async_copy(u_hbm.at[sidx_v.at[j0 + 2]], rows0_v, sem0)

        pltpu.make_async_copy(u_hbm.at[sidx_v.at[j0 + 1]], rows1_v,
                              sem1).wait()
        pltpu.sync_copy(rows1_v, acc_sh.at[didx_v.at[j0 + 1]], add=True)

        @pl.when(g + 1 < CHUNKS // 2)
        def _():
            pltpu.async_copy(u_hbm.at[sidx_v.at[j0 + 3]], rows1_v, sem1)

        return carry

    lax.fori_loop(0, CHUNKS // 2, body, 0)
    plsc.subcore_barrier()

    pltpu.sync_copy(acc_sh.at[pl.ds(s * RPT, RPT)], stage_v)
    pltpu.sync_copy(stage_v, out_hbm.at[c, pl.ds(s * RPT, RPT)])


_sc_agg = functools.partial(
    pl.kernel,
    out_type=jax.ShapeDtypeStruct((NC, NP, H), jnp.float32),
    mesh=plsc.VectorSubcoreMesh(core_axis_name="c", subcore_axis_name="s",
                                num_cores=NC, num_subcores=NS),
    scratch_types=[
        pltpu.VMEM((CHUNKS, EB), jnp.int32),
        pltpu.VMEM((CHUNKS, EB), jnp.int32),
        pltpu.VMEM((EB, H), jnp.float32),
        pltpu.VMEM((EB, H), jnp.float32),
        pltpu.VMEM((RPT, H), jnp.float32),
        pltpu.VMEM_SHARED((NP, H), jnp.float32),
        pltpu.SemaphoreType.DMA,
        pltpu.SemaphoreType.DMA,
    ],
    compiler_params=pltpu.CompilerParams(use_tc_tiling_on_sc=False),
)(_sc_agg_body)


def _mm_body(x_ref, w_ref, o_ref):
    o_ref[...] = jnp.dot(x_ref[...], w_ref[...],
                         preferred_element_type=jnp.float32, precision=jax.lax.Precision.HIGHEST)


def _conv_post_body(u_ref, a0_ref, a1_ref, ba_ref, wb_ref, bb_ref,
                    h_ref, s_ref, s2_ref):
    z = jnp.maximum(u_ref[...] + a0_ref[...] + a1_ref[...] + ba_ref[...], 0.0)
    h = jnp.dot(z, wb_ref[...], preferred_element_type=jnp.float32, precision=jax.lax.Precision.HIGHEST) + bb_ref[...]
    h_ref[...] = h

    @pl.when(pl.program_id(0) == 0)
    def _():
        s_ref[...] = jnp.zeros_like(s_ref)
        s2_ref[...] = jnp.zeros_like(s2_ref)

    hr = h.reshape(BN_ROWS // 8, 8, H)
    s_ref[...] += jnp.sum(hr, axis=0)
    s2_ref[...] += jnp.sum(hr * hr, axis=0)


def _bn_mm_body(h_ref, s_ref, s2_ref, g_ref, be_ref, w_ref, o_ref):
    sm = jnp.sum(s_ref[...], axis=0, keepdims=True)
    sq = jnp.sum(s2_ref[...], axis=0, keepdims=True)
    m = sm / N
    var = sq / N - m * m
    scale = g_ref[...] * jax.lax.rsqrt(var + 1e-5)
    shift = be_ref[...] - m * scale
    hn = h_ref[...] * scale + shift
    o_ref[...] = jnp.dot(hn, w_ref[...], preferred_element_type=jnp.float32, precision=jax.lax.Precision.HIGHEST)


def _head_body(h_ref, s_ref, s2_ref, g_ref, be_ref, wf1_ref, bf1_ref,
               wf2_ref, bf2_ref, o_ref):
    sm = jnp.sum(s_ref[...], axis=0, keepdims=True)
    sq = jnp.sum(s2_ref[...], axis=0, keepdims=True)
    m = sm / N
    var = sq / N - m * m
    scale = g_ref[...] * jax.lax.rsqrt(var + 1e-5)
    shift = be_ref[...] - m * scale
    hn = h_ref[...] * scale + shift
    f = jnp.maximum(
        jnp.dot(hn, wf1_ref[...], preferred_element_type=jnp.float32, precision=jax.lax.Precision.HIGHEST)
        + bf1_ref[...], 0.0)
    o_ref[...] = jnp.dot(f, wf2_ref[...],
                         preferred_element_type=jnp.float32, precision=jax.lax.Precision.HIGHEST) + bf2_ref[...]


def _row_spec(width):
    return pl.BlockSpec((BN_ROWS, width), lambda i: (i, 0))


def _full_spec(shape):
    return pl.BlockSpec(shape, lambda i: tuple(0 for _ in shape))


def _mm(x, w, in_width, out_width):
    return pl.pallas_call(
        _mm_body,
        grid=(GRID,),
        in_specs=[_row_spec(in_width), _full_spec(w.shape)],
        out_specs=_row_spec(out_width),
        out_shape=jax.ShapeDtypeStruct((N, out_width), jnp.float32),
    )(x, w)


def _conv_post(u, agg, ba, wb, bb):
    return pl.pallas_call(
        _conv_post_body,
        grid=(GRID,),
        in_specs=[_row_spec(H), _row_spec(H), _row_spec(H),
                  _full_spec((1, H)), _full_spec((H, H)), _full_spec((1, H))],
        out_specs=[_row_spec(H), _full_spec((8, H)), _full_spec((8, H))],
        out_shape=[jax.ShapeDtypeStruct((N, H), jnp.float32),
                   jax.ShapeDtypeStruct((8, H), jnp.float32),
                   jax.ShapeDtypeStruct((8, H), jnp.float32)],
    )(u, agg[0], agg[1], ba.reshape(1, H), wb, bb.reshape(1, H))


def _bn_mm(h, s, s2, g, be, w):
    return pl.pallas_call(
        _bn_mm_body,
        grid=(GRID,),
        in_specs=[_row_spec(H), _full_spec((8, H)), _full_spec((8, H)),
                  _full_spec((1, H)), _full_spec((1, H)), _full_spec((H, H))],
        out_specs=_row_spec(H),
        out_shape=jax.ShapeDtypeStruct((N, H), jnp.float32),
    )(h, s, s2, g.reshape(1, H), be.reshape(1, H), w)


def _head(h, s, s2, g, be, wf1, bf1, wf2, bf2):
    return pl.pallas_call(
        _head_body,
        grid=(GRID,),
        in_specs=[_row_spec(H), _full_spec((8, H)), _full_spec((8, H)),
                  _full_spec((1, H)), _full_spec((1, H)),
                  _full_spec((H, H)), _full_spec((1, H)),
                  _full_spec((H, C)), _full_spec((1, C))],
        out_specs=_row_spec(C),
        out_shape=jax.ShapeDtypeStruct((N, C), jnp.float32),
    )(h, s, s2, g.reshape(1, H), be.reshape(1, H),
      wf1, bf1.reshape(1, H), wf2, bf2.reshape(1, C))


def kernel(x, edge_index, W1a, b1a, W1b, b1b, g1, be1,
           W2a, b2a, W2b, b2b, g2, be2, Wf1, bf1, Wf2, bf2):
    src = edge_index[0]
    dst = edge_index[1]
    ppw = EPW_PAD - EPW
    w = jnp.arange(NW, dtype=jnp.int32)[:, None]
    dsrc = (w * 311 + jnp.arange(ppw, dtype=jnp.int32)[None, :] * 13) % N
    src_p = jnp.concatenate(
        [src.reshape(NW, EPW), dsrc], axis=1).reshape(NW, CHUNKS, EB)
    dum = N + (w % NS) * 7 + jnp.arange(ppw, dtype=jnp.int32)[None, :] % 7
    dst_p = jnp.concatenate(
        [dst.reshape(NW, EPW), dum], axis=1).reshape(NW, CHUNKS, EB)
    zeros = jnp.zeros((NP, H), jnp.float32)

    u = _mm(x, W1a, D, H)
    agg = _sc_agg(u, src_p, dst_p, zeros)
    h1, s1, s1sq = _conv_post(u, agg, b1a, W1b, b1b)
    v2 = _bn_mm(h1, s1, s1sq, g1, be1, W2a)
    agg2 = _sc_agg(v2, src_p, dst_p, zeros)
    h2, s2, s2sq = _conv_post(v2, agg2, b2a, W2b, b2b)
    return _head(h2, s2, s2sq, g2, be2, Wf1, bf1, Wf2, bf2)


# speedup vs baseline: 1.7806x; 1.1212x over previous
import functools

import jax
import jax.numpy as jnp
from jax import lax
from jax.experimental import pallas as pl
from jax.experimental.pallas import tpu as pltpu
from jax.experimental.pallas import tpu_sc as plsc

N = 10000
E = 320000
D = 128
H = 32
C = 40

NC = 2
NS = 16
NW = NC * NS
EB = 128
EPW = E // NW
CHUNKS = 80
EPW_PAD = CHUNKS * EB
NP = 10112
RPT = NP // NS

BN_ROWS = 2000
GRID = N // BN_ROWS


NB = 4
GROUPS = CHUNKS // NB


def _sc_agg_body(u_hbm, src_hbm, dst_hbm, zeros_hbm, out_hbm,
                 sidx_v, didx_v, rows_v, stage_v, acc_sh, gsems, ssems):
    c = lax.axis_index("c")
    s = lax.axis_index("s")
    wid = c * NS + s

    pltpu.sync_copy(zeros_hbm.at[pl.ds(s * RPT, RPT)], stage_v)
    pltpu.sync_copy(stage_v, acc_sh.at[pl.ds(s * RPT, RPT)])

    pltpu.sync_copy(src_hbm.at[wid], sidx_v)
    pltpu.sync_copy(dst_hbm.at[wid], didx_v)
    plsc.subcore_barrier()

    for b in range(NB):
        pltpu.async_copy(u_hbm.at[sidx_v.at[b]], rows_v.at[b], gsems.at[b])

    def body(g, carry):
        j = NB * g
        for b in range(NB):
            pltpu.make_async_copy(u_hbm.at[sidx_v.at[j + b]], rows_v.at[b],
                                  gsems.at[b]).wait()
            pltpu.async_copy(rows_v.at[b], acc_sh.at[didx_v.at[j + b]],
                             ssems.at[b], add=True)
        for b in range(NB):
            pltpu.make_async_copy(rows_v.at[b], acc_sh.at[didx_v.at[j + b]],
                                  ssems.at[b]).wait()

            @pl.when(g + 1 < GROUPS)
            def _():
                pltpu.async_copy(u_hbm.at[sidx_v.at[j + NB + b]],
                                 rows_v.at[b], gsems.at[b])

        return carry

    lax.fori_loop(0, GROUPS, body, 0)
    plsc.subcore_barrier()

    pltpu.sync_copy(acc_sh.at[pl.ds(s * RPT, RPT)], stage_v)
    pltpu.sync_copy(stage_v, out_hbm.at[c, pl.ds(s * RPT, RPT)])


_sc_agg = functools.partial(
    pl.kernel,
    out_type=jax.ShapeDtypeStruct((NC, NP, H), jnp.float32),
    mesh=plsc.VectorSubcoreMesh(core_axis_name="c", subcore_axis_name="s",
                                num_cores=NC, num_subcores=NS),
    scratch_types=[
        pltpu.VMEM((CHUNKS, EB), jnp.int32),
        pltpu.VMEM((CHUNKS, EB), jnp.int32),
        pltpu.VMEM((NB, EB, H), jnp.float32),
        pltpu.VMEM((RPT, H), jnp.float32),
        pltpu.VMEM_SHARED((NP, H), jnp.float32),
        pltpu.SemaphoreType.DMA((NB,)),
        pltpu.SemaphoreType.DMA((NB,)),
    ],
    compiler_params=pltpu.CompilerParams(use_tc_tiling_on_sc=False),
)(_sc_agg_body)


def _mm_body(x_ref, w_ref, o_ref):
    o_ref[...] = jnp.dot(x_ref[...], w_ref[...],
                         preferred_element_type=jnp.float32, precision=jax.lax.Precision.HIGHEST)


def _conv_post_body(u_ref, a0_ref, a1_ref, ba_ref, wb_ref, bb_ref,
                    h_ref, s_ref, s2_ref):
    z = jnp.maximum(u_ref[...] + a0_ref[...] + a1_ref[...] + ba_ref[...], 0.0)
    h = jnp.dot(z, wb_ref[...], preferred_element_type=jnp.float32, precision=jax.lax.Precision.HIGHEST) + bb_ref[...]
    h_ref[...] = h

    @pl.when(pl.program_id(0) == 0)
    def _():
        s_ref[...] = jnp.zeros_like(s_ref)
        s2_ref[...] = jnp.zeros_like(s2_ref)

    hr = h.reshape(BN_ROWS // 8, 8, H)
    s_ref[...] += jnp.sum(hr, axis=0)
    s2_ref[...] += jnp.sum(hr * hr, axis=0)


def _bn_mm_body(h_ref, s_ref, s2_ref, g_ref, be_ref, w_ref, o_ref):
    sm = jnp.sum(s_ref[...], axis=0, keepdims=True)
    sq = jnp.sum(s2_ref[...], axis=0, keepdims=True)
    m = sm / N
    var = sq / N - m * m
    scale = g_ref[...] * jax.lax.rsqrt(var + 1e-5)
    shift = be_ref[...] - m * scale
    hn = h_ref[...] * scale + shift
    o_ref[...] = jnp.dot(hn, w_ref[...], preferred_element_type=jnp.float32, precision=jax.lax.Precision.HIGHEST)


def _head_body(h_ref, s_ref, s2_ref, g_ref, be_ref, wf1_ref, bf1_ref,
               wf2_ref, bf2_ref, o_ref):
    sm = jnp.sum(s_ref[...], axis=0, keepdims=True)
    sq = jnp.sum(s2_ref[...], axis=0, keepdims=True)
    m = sm / N
    var = sq / N - m * m
    scale = g_ref[...] * jax.lax.rsqrt(var + 1e-5)
    shift = be_ref[...] - m * scale
    hn = h_ref[...] * scale + shift
    f = jnp.maximum(
        jnp.dot(hn, wf1_ref[...], preferred_element_type=jnp.float32, precision=jax.lax.Precision.HIGHEST)
        + bf1_ref[...], 0.0)
    o_ref[...] = jnp.dot(f, wf2_ref[...],
                         preferred_element_type=jnp.float32, precision=jax.lax.Precision.HIGHEST) + bf2_ref[...]


def _row_spec(width):
    return pl.BlockSpec((BN_ROWS, width), lambda i: (i, 0))


def _full_spec(shape):
    return pl.BlockSpec(shape, lambda i: tuple(0 for _ in shape))


def _mm(x, w, in_width, out_width):
    return pl.pallas_call(
        _mm_body,
        grid=(GRID,),
        in_specs=[_row_spec(in_width), _full_spec(w.shape)],
        out_specs=_row_spec(out_width),
        out_shape=jax.ShapeDtypeStruct((N, out_width), jnp.float32),
    )(x, w)


def _conv_post(u, agg, ba, wb, bb):
    return pl.pallas_call(
        _conv_post_body,
        grid=(GRID,),
        in_specs=[_row_spec(H), _row_spec(H), _row_spec(H),
                  _full_spec((1, H)), _full_spec((H, H)), _full_spec((1, H))],
        out_specs=[_row_spec(H), _full_spec((8, H)), _full_spec((8, H))],
        out_shape=[jax.ShapeDtypeStruct((N, H), jnp.float32),
                   jax.ShapeDtypeStruct((8, H), jnp.float32),
                   jax.ShapeDtypeStruct((8, H), jnp.float32)],
    )(u, agg[0], agg[1], ba.reshape(1, H), wb, bb.reshape(1, H))


def _bn_mm(h, s, s2, g, be, w):
    return pl.pallas_call(
        _bn_mm_body,
        grid=(GRID,),
        in_specs=[_row_spec(H), _full_spec((8, H)), _full_spec((8, H)),
                  _full_spec((1, H)), _full_spec((1, H)), _full_spec((H, H))],
        out_specs=_row_spec(H),
        out_shape=jax.ShapeDtypeStruct((N, H), jnp.float32),
    )(h, s, s2, g.reshape(1, H), be.reshape(1, H), w)


def _head(h, s, s2, g, be, wf1, bf1, wf2, bf2):
    return pl.pallas_call(
        _head_body,
        grid=(GRID,),
        in_specs=[_row_spec(H), _full_spec((8, H)), _full_spec((8, H)),
                  _full_spec((1, H)), _full_spec((1, H)),
                  _full_spec((H, H)), _full_spec((1, H)),
                  _full_spec((H, C)), _full_spec((1, C))],
        out_specs=_row_spec(C),
        out_shape=jax.ShapeDtypeStruct((N, C), jnp.float32),
    )(h, s, s2, g.reshape(1, H), be.reshape(1, H),
      wf1, bf1.reshape(1, H), wf2, bf2.reshape(1, C))


def kernel(x, edge_index, W1a, b1a, W1b, b1b, g1, be1,
           W2a, b2a, W2b, b2b, g2, be2, Wf1, bf1, Wf2, bf2):
    src = edge_index[0]
    dst = edge_index[1]
    ppw = EPW_PAD - EPW
    w = jnp.arange(NW, dtype=jnp.int32)[:, None]
    dsrc = (w * 311 + jnp.arange(ppw, dtype=jnp.int32)[None, :] * 13) % N
    src_p = jnp.concatenate(
        [src.reshape(NW, EPW), dsrc], axis=1).reshape(NW, CHUNKS, EB)
    dum = N + (w % NS) * 7 + jnp.arange(ppw, dtype=jnp.int32)[None, :] % 7
    dst_p = jnp.concatenate(
        [dst.reshape(NW, EPW), dum], axis=1).reshape(NW, CHUNKS, EB)
    zeros = jnp.zeros((NP, H), jnp.float32)

    u = _mm(x, W1a, D, H)
    agg = _sc_agg(u, src_p, dst_p, zeros)
    h1, s1, s1sq = _conv_post(u, agg, b1a, W1b, b1b)
    v2 = _bn_mm(h1, s1, s1sq, g1, be1, W2a)
    agg2 = _sc_agg(v2, src_p, dst_p, zeros)
    h2, s2, s2sq = _conv_post(v2, agg2, b2a, W2b, b2b)
    return _head(h2, s2, s2sq, g2, be2, Wf1, bf1, Wf2, bf2)


# speedup vs baseline: 2.5551x; 1.4349x over previous
import functools

import jax
import jax.numpy as jnp
from jax import lax
from jax.experimental import pallas as pl
from jax.experimental.pallas import tpu as pltpu
from jax.experimental.pallas import tpu_sc as plsc

N = 10000
E = 320000
D = 128
H = 32
C = 40

NC = 2
NS = 16
NW = NC * NS
EB = 80
EPW = E // NW
CHUNKS = EPW // EB
NB = 5
GROUPS = CHUNKS // NB
NP = 10112
RPT = NP // NS

PK = 4
PN = N // PK
PNP = NP // PK
PH = PK * H
PR = 512
GRID = -(-PN // PR)

_PREC = jax.lax.Precision.HIGHEST


def _sc_agg_body(u_hbm, eidx_hbm, zeros_hbm, out0_hbm, out1_hbm,
                 sidx_v, didx_v, rows_v, stage_v, acc_sh, gsems, ssems):
    c = lax.axis_index("c")
    s = lax.axis_index("s")
    wid = c * NS + s

    pltpu.sync_copy(zeros_hbm.at[pl.ds(s * RPT, RPT)], stage_v)
    pltpu.sync_copy(stage_v, acc_sh.at[pl.ds(s * RPT, RPT)])

    pltpu.sync_copy(eidx_hbm.at[0, wid], sidx_v)
    pltpu.sync_copy(eidx_hbm.at[1, wid], didx_v)
    plsc.subcore_barrier()

    for b in range(NB):
        pltpu.async_copy(u_hbm.at[sidx_v.at[b]], rows_v.at[b], gsems.at[b])

    def body(g, carry):
        j = NB * g
        for b in range(NB):
            pltpu.make_async_copy(u_hbm.at[sidx_v.at[j + b]], rows_v.at[b],
                                  gsems.at[b]).wait()
            pltpu.async_copy(rows_v.at[b], acc_sh.at[didx_v.at[j + b]],
                             ssems.at[b], add=True)
        for b in range(NB):
            pltpu.make_async_copy(rows_v.at[b], acc_sh.at[didx_v.at[j + b]],
                                  ssems.at[b]).wait()

            @pl.when(g + 1 < GROUPS)
            def _():
                pltpu.async_copy(u_hbm.at[sidx_v.at[j + NB + b]],
                                 rows_v.at[b], gsems.at[b])

        return carry

    lax.fori_loop(0, GROUPS, body, 0)
    plsc.subcore_barrier()

    pltpu.sync_copy(acc_sh.at[pl.ds(s * RPT, RPT)], stage_v)

    @pl.when(c == 0)
    def _():
        pltpu.sync_copy(stage_v, out0_hbm.at[pl.ds(s * RPT, RPT)])

    @pl.when(c == 1)
    def _():
        pltpu.sync_copy(stage_v, out1_hbm.at[pl.ds(s * RPT, RPT)])


_sc_agg = functools.partial(
    pl.kernel,
    out_type=[jax.ShapeDtypeStruct((NP, H), jnp.float32),
              jax.ShapeDtypeStruct((NP, H), jnp.float32)],
    mesh=plsc.VectorSubcoreMesh(core_axis_name="c", subcore_axis_name="s",
                                num_cores=NC, num_subcores=NS),
    scratch_types=[
        pltpu.VMEM((CHUNKS, EB), jnp.int32),
        pltpu.VMEM((CHUNKS, EB), jnp.int32),
        pltpu.VMEM((NB, EB, H), jnp.float32),
        pltpu.VMEM((RPT, H), jnp.float32),
        pltpu.VMEM_SHARED((NP, H), jnp.float32),
        pltpu.SemaphoreType.DMA((NB,)),
        pltpu.SemaphoreType.DMA((NB,)),
    ],
    compiler_params=pltpu.CompilerParams(use_tc_tiling_on_sc=False),
)(_sc_agg_body)


def _mm_body(x_ref, w_ref, o_ref):
    o_ref[...] = jnp.dot(x_ref[...], w_ref[...],
                         preferred_element_type=jnp.float32, precision=_PREC)


def _stats(h, i, s_ref, s2_ref):
    @pl.when(i == 0)
    def _():
        s_ref[...] = jnp.zeros_like(s_ref)
        s2_ref[...] = jnp.zeros_like(s2_ref)

    ri = jax.lax.broadcasted_iota(jnp.int32, (PR, 1), 0) + i * PR
    hm = jnp.where(ri < PN, h, 0.0)
    hr = hm.reshape(PR // 8, 8, PH)
    s_ref[...] += jnp.sum(hr, axis=0)
    s2_ref[...] += jnp.sum(hr * hr, axis=0)


def _conv_post_body(u_ref, a0_ref, a1_ref, ba_ref, wbd_ref, bb_ref,
                    h_ref, s_ref, s2_ref):
    z = jnp.maximum(u_ref[...] + a0_ref[...] + a1_ref[...] + ba_ref[...], 0.0)
    h = jnp.dot(z, wbd_ref[...], preferred_element_type=jnp.float32,
                precision=_PREC) + bb_ref[...]
    h_ref[...] = h
    _stats(h, pl.program_id(0), s_ref, s2_ref)


def _bn_scale_shift(s_ref, s2_ref, g_ref, be_ref, summ_ref):
    summ = summ_ref[...]
    s32 = jnp.dot(jnp.sum(s_ref[...], axis=0, keepdims=True), summ,
                  preferred_element_type=jnp.float32, precision=_PREC)
    q32 = jnp.dot(jnp.sum(s2_ref[...], axis=0, keepdims=True), summ,
                  preferred_element_type=jnp.float32, precision=_PREC)
    m = s32 / N
    var = q32 / N - m * m
    scale = g_ref[...] * jax.lax.rsqrt(var + 1e-5)
    shift = be_ref[...] - m * scale
    rep = summ.T
    scale128 = jnp.dot(scale, rep, preferred_element_type=jnp.float32,
                       precision=_PREC)
    shift128 = jnp.dot(shift, rep, preferred_element_type=jnp.float32,
                       precision=_PREC)
    return scale128, shift128


def _bn_mm_body(h_ref, s_ref, s2_ref, g_ref, be_ref, summ_ref, wbd_ref,
                o_ref):
    scale128, shift128 = _bn_scale_shift(s_ref, s2_ref, g_ref, be_ref,
                                         summ_ref)
    hn = h_ref[...] * scale128 + shift128
    o_ref[...] = jnp.dot(hn, wbd_ref[...], preferred_element_type=jnp.float32,
                         precision=_PREC)


def _head_body(h_ref, s_ref, s2_ref, g_ref, be_ref, summ_ref,
               wf1d_ref, bf1_ref, wf2d_ref, bf2_ref, o_ref):
    scale128, shift128 = _bn_scale_shift(s_ref, s2_ref, g_ref, be_ref,
                                         summ_ref)
    hn = h_ref[...] * scale128 + shift128
    f = jnp.maximum(
        jnp.dot(hn, wf1d_ref[...], preferred_element_type=jnp.float32,
                precision=_PREC) + bf1_ref[...], 0.0)
    o_ref[...] = jnp.dot(f, wf2d_ref[...], preferred_element_type=jnp.float32,
                         precision=_PREC) + bf2_ref[...]


def _row_spec(width):
    return pl.BlockSpec((PR, width), lambda i: (i, 0))


def _full_spec(shape):
    return pl.BlockSpec(shape, lambda i: tuple(0 for _ in shape))


def _blockdiag(w):
    h, wdt = w.shape
    z = jnp.zeros((h, wdt), jnp.float32)
    rows = [jnp.concatenate([w if j == k else z for j in range(PK)], axis=1)
            for k in range(PK)]
    return jnp.concatenate(rows, axis=0)


def _mm(x4, wbd):
    return pl.pallas_call(
        _mm_body,
        grid=(GRID,),
        in_specs=[_row_spec(PK * D), _full_spec(wbd.shape)],
        out_specs=_row_spec(PH),
        out_shape=jax.ShapeDtypeStruct((PN, PH), jnp.float32),
    )(x4, wbd)


def _conv_post(u4, a0, a1, ba4, wbd, bb4):
    return pl.pallas_call(
        _conv_post_body,
        grid=(GRID,),
        in_specs=[_row_spec(PH), _row_spec(PH), _row_spec(PH),
                  _full_spec((1, PH)), _full_spec((PH, PH)),
                  _full_spec((1, PH))],
        out_specs=[_row_spec(PH), _full_spec((8, PH)), _full_spec((8, PH))],
        out_shape=[jax.ShapeDtypeStruct((PN, PH), jnp.float32),
                   jax.ShapeDtypeStruct((8, PH), jnp.float32),
                   jax.ShapeDtypeStruct((8, PH), jnp.float32)],
    )(u4, a0, a1, ba4, wbd, bb4)


def _bn_mm(h4, s, s2, g, be, summ, wbd):
    return pl.pallas_call(
        _bn_mm_body,
        grid=(GRID,),
        in_specs=[_row_spec(PH), _full_spec((8, PH)), _full_spec((8, PH)),
                  _full_spec((1, H)), _full_spec((1, H)),
                  _full_spec((PH, H)), _full_spec((PH, PH))],
        out_specs=_row_spec(PH),
        out_shape=jax.ShapeDtypeStruct((PN, PH), jnp.float32),
    )(h4, s, s2, g.reshape(1, H), be.reshape(1, H), summ, wbd)


def _head(h4, s, s2, g, be, summ, wf1d, bf1_4, wf2d, bf2_4):
    return pl.pallas_call(
        _head_body,
        grid=(GRID,),
        in_specs=[_row_spec(PH), _full_spec((8, PH)), _full_spec((8, PH)),
                  _full_spec((1, H)), _full_spec((1, H)),
                  _full_spec((PH, H)), _full_spec((PH, PH)),
                  _full_spec((1, PH)), _full_spec((PH, PK * C)),
                  _full_spec((1, PK * C))],
        out_specs=_row_spec(PK * C),
        out_shape=jax.ShapeDtypeStruct((PN, PK * C), jnp.float32),
    )(h4, s, s2, g.reshape(1, H), be.reshape(1, H), summ,
      wf1d, bf1_4, wf2d, bf2_4)


def kernel(x, edge_index, W1a, b1a, W1b, b1b, g1, be1,
           W2a, b2a, W2b, b2b, g2, be2, Wf1, bf1, Wf2, bf2):
    eidx = edge_index.reshape(2, NW, CHUNKS, EB)
    zeros = jnp.zeros((NP, H), jnp.float32)
    summ = jnp.tile(jnp.eye(H, dtype=jnp.float32), (PK, 1))

    w1ad = _blockdiag(W1a)
    w1bd = _blockdiag(W1b)
    w2ad = _blockdiag(W2a)
    w2bd = _blockdiag(W2b)
    wf1d = _blockdiag(Wf1)
    wf2d = _blockdiag(Wf2)
    b1a4 = jnp.tile(b1a, PK).reshape(1, PH)
    b1b4 = jnp.tile(b1b, PK).reshape(1, PH)
    b2a4 = jnp.tile(b2a, PK).reshape(1, PH)
    b2b4 = jnp.tile(b2b, PK).reshape(1, PH)
    bf14 = jnp.tile(bf1, PK).reshape(1, PH)
    bf24 = jnp.tile(bf2, PK).reshape(1, PK * C)

    u4 = _mm(x.reshape(PN, PK * D), w1ad)
    agg0, agg1 = _sc_agg(u4.reshape(N, H), eidx, zeros)
    h1, s1, s1q = _conv_post(u4, agg0.reshape(PNP, PH), agg1.reshape(PNP, PH),
                             b1a4, w1bd, b1b4)
    v2 = _bn_mm(h1, s1, s1q, g1, be1, summ, w2ad)
    agg0b, agg1b = _sc_agg(v2.reshape(N, H), eidx, zeros)
    h2, s2, s2q = _conv_post(v2, agg0b.reshape(PNP, PH),
                             agg1b.reshape(PNP, PH), b2a4, w2bd, b2b4)
    out4 = _head(h2, s2, s2q, g2, be2, summ, wf1d, bf14, wf2d, bf24)
    return out4.reshape(N, C)


# speedup vs baseline: 2.8225x; 1.1047x over previous
import functools

import jax
import jax.numpy as jnp
from jax import lax
from jax.experimental import pallas as pl
from jax.experimental.pallas import tpu as pltpu
from jax.experimental.pallas import tpu_sc as plsc

N = 10000
E = 320000
D = 128
H = 32
C = 40

NC = 2
NS = 16
NW = NC * NS
EB = 80
EPW = E // NW
CHUNKS = EPW // EB
NB = 5
HGROUPS = (CHUNKS // NB - 1) // 2
NP = 10112
RPT = NP // NS

PK = 4
PN = N // PK
PNP = NP // PK
PH = PK * H
PR = 512
GRID = -(-PN // PR)

_PREC = jax.lax.Precision.HIGHEST


def _sc_agg_body(u_hbm, eidx_hbm, zeros_hbm, out0_hbm, out1_hbm,
                 sidx_v, didx_v, rows_v, stage_v, acc_sh, gsems, ssems):
    c = lax.axis_index("c")
    s = lax.axis_index("s")
    wid = c * NS + s

    pltpu.sync_copy(zeros_hbm.at[pl.ds(s * RPT, RPT)], stage_v)
    pltpu.sync_copy(stage_v, acc_sh.at[pl.ds(s * RPT, RPT)])

    pltpu.sync_copy(eidx_hbm.at[0, wid], sidx_v)
    pltpu.sync_copy(eidx_hbm.at[1, wid], didx_v)
    plsc.subcore_barrier()

    def _gather(j, bank, b):
        pltpu.async_copy(u_hbm.at[sidx_v.at[j]], rows_v.at[bank, b],
                         gsems.at[bank, b])

    def _wait_gather(j, bank, b):
        pltpu.make_async_copy(u_hbm.at[sidx_v.at[j]], rows_v.at[bank, b],
                              gsems.at[bank, b]).wait()

    def _scatter(j, bank, b):
        pltpu.async_copy(rows_v.at[bank, b], acc_sh.at[didx_v.at[j]],
                         ssems.at[bank, b], add=True)

    def _wait_scatter(j, bank, b):
        pltpu.make_async_copy(rows_v.at[bank, b], acc_sh.at[didx_v.at[j]],
                              ssems.at[bank, b]).wait()

    for b in range(NB):
        _gather(b, 0, b)
    for b in range(NB):
        _gather(NB + b, 1, b)

    def body(t, carry):
        ja = 2 * NB * t
        jb = ja + NB
        for b in range(NB):
            _wait_gather(ja + b, 0, b)
            _scatter(ja + b, 0, b)
        for b in range(NB):
            _wait_scatter(ja + b, 0, b)
            _gather(ja + 2 * NB + b, 0, b)
        for b in range(NB):
            _wait_gather(jb + b, 1, b)
            _scatter(jb + b, 1, b)
        for b in range(NB):
            _wait_scatter(jb + b, 1, b)

            @pl.when(t + 1 < HGROUPS)
            def _():
                _gather(jb + 2 * NB + b, 1, b)

        return carry

    lax.fori_loop(0, HGROUPS, body, 0)

    jt = 2 * NB * HGROUPS
    for b in range(NB):
        _wait_gather(jt + b, 0, b)
        _scatter(jt + b, 0, b)
    for b in range(NB):
        _wait_scatter(jt + b, 0, b)
    plsc.subcore_barrier()

    pltpu.sync_copy(acc_sh.at[pl.ds(s * RPT, RPT)], stage_v)

    @pl.when(c == 0)
    def _():
        pltpu.sync_copy(stage_v, out0_hbm.at[pl.ds(s * RPT, RPT)])

    @pl.when(c == 1)
    def _():
        pltpu.sync_copy(stage_v, out1_hbm.at[pl.ds(s * RPT, RPT)])


_sc_agg = functools.partial(
    pl.kernel,
    out_type=[jax.ShapeDtypeStruct((NP, H), jnp.float32),
              jax.ShapeDtypeStruct((NP, H), jnp.float32)],
    mesh=plsc.VectorSubcoreMesh(core_axis_name="c", subcore_axis_name="s",
                                num_cores=NC, num_subcores=NS),
    scratch_types=[
        pltpu.VMEM((CHUNKS, EB), jnp.int32),
        pltpu.VMEM((CHUNKS, EB), jnp.int32),
        pltpu.VMEM((2, NB, EB, H), jnp.float32),
        pltpu.VMEM((RPT, H), jnp.float32),
        pltpu.VMEM_SHARED((NP, H), jnp.float32),
        pltpu.SemaphoreType.DMA((2, NB)),
        pltpu.SemaphoreType.DMA((2, NB)),
    ],
    compiler_params=pltpu.CompilerParams(use_tc_tiling_on_sc=False),
)(_sc_agg_body)


def _mm_body(x_ref, w_ref, o_ref):
    o_ref[...] = jnp.dot(x_ref[...], w_ref[...],
                         preferred_element_type=jnp.float32, precision=_PREC)


def _stats(h, i, s_ref, s2_ref):
    @pl.when(i == 0)
    def _():
        s_ref[...] = jnp.zeros_like(s_ref)
        s2_ref[...] = jnp.zeros_like(s2_ref)

    ri = jax.lax.broadcasted_iota(jnp.int32, (PR, 1), 0) + i * PR
    hm = jnp.where(ri < PN, h, 0.0)
    hr = hm.reshape(PR // 8, 8, PH)
    s_ref[...] += jnp.sum(hr, axis=0)
    s2_ref[...] += jnp.sum(hr * hr, axis=0)


def _conv_post_body(u_ref, a0_ref, a1_ref, ba_ref, wbd_ref, bb_ref,
                    h_ref, s_ref, s2_ref):
    z = jnp.maximum(u_ref[...] + a0_ref[...] + a1_ref[...] + ba_ref[...], 0.0)
    h = jnp.dot(z, wbd_ref[...], preferred_element_type=jnp.float32,
                precision=_PREC) + bb_ref[...]
    h_ref[...] = h
    _stats(h, pl.program_id(0), s_ref, s2_ref)


def _bn_scale_shift(s_ref, s2_ref, g_ref, be_ref, summ_ref):
    summ = summ_ref[...]
    s32 = jnp.dot(jnp.sum(s_ref[...], axis=0, keepdims=True), summ,
                  preferred_element_type=jnp.float32, precision=_PREC)
    q32 = jnp.dot(jnp.sum(s2_ref[...], axis=0, keepdims=True), summ,
                  preferred_element_type=jnp.float32, precision=_PREC)
    m = s32 / N
    var = q32 / N - m * m
    scale = g_ref[...] * jax.lax.rsqrt(var + 1e-5)
    shift = be_ref[...] - m * scale
    rep = summ.T
    scale128 = jnp.dot(scale, rep, preferred_element_type=jnp.float32,
                       precision=_PREC)
    shift128 = jnp.dot(shift, rep, preferred_element_type=jnp.float32,
                       precision=_PREC)
    return scale128, shift128


def _bn_mm_body(h_ref, s_ref, s2_ref, g_ref, be_ref, summ_ref, wbd_ref,
                o_ref):
    scale128, shift128 = _bn_scale_shift(s_ref, s2_ref, g_ref, be_ref,
                                         summ_ref)
    hn = h_ref[...] * scale128 + shift128
    o_ref[...] = jnp.dot(hn, wbd_ref[...], preferred_element_type=jnp.float32,
                         precision=_PREC)


def _head_body(h_ref, s_ref, s2_ref, g_ref, be_ref, summ_ref,
               wf1d_ref, bf1_ref, wf2d_ref, bf2_ref, o_ref):
    scale128, shift128 = _bn_scale_shift(s_ref, s2_ref, g_ref, be_ref,
                                         summ_ref)
    hn = h_ref[...] * scale128 + shift128
    f = jnp.maximum(
        jnp.dot(hn, wf1d_ref[...], preferred_element_type=jnp.float32,
                precision=_PREC) + bf1_ref[...], 0.0)
    o_ref[...] = jnp.dot(f, wf2d_ref[...], preferred_element_type=jnp.float32,
                         precision=_PREC) + bf2_ref[...]


def _row_spec(width):
    return pl.BlockSpec((PR, width), lambda i: (i, 0))


def _full_spec(shape):
    return pl.BlockSpec(shape, lambda i: tuple(0 for _ in shape))


def _blockdiag(w):
    h, wdt = w.shape
    z = jnp.zeros((h, wdt), jnp.float32)
    rows = [jnp.concatenate([w if j == k else z for j in range(PK)], axis=1)
            for k in range(PK)]
    return jnp.concatenate(rows, axis=0)


def _mm(x4, wbd):
    return pl.pallas_call(
        _mm_body,
        grid=(GRID,),
        in_specs=[_row_spec(PK * D), _full_spec(wbd.shape)],
        out_specs=_row_spec(PH),
        out_shape=jax.ShapeDtypeStruct((PN, PH), jnp.float32),
    )(x4, wbd)


def _conv_post(u4, a0, a1, ba4, wbd, bb4):
    return pl.pallas_call(
        _conv_post_body,
        grid=(GRID,),
        in_specs=[_row_spec(PH), _row_spec(PH), _row_spec(PH),
                  _full_spec((1, PH)), _full_spec((PH, PH)),
                  _full_spec((1, PH))],
        out_specs=[_row_spec(PH), _full_spec((8, PH)), _full_spec((8, PH))],
        out_shape=[jax.ShapeDtypeStruct((PN, PH), jnp.float32),
                   jax.ShapeDtypeStruct((8, PH), jnp.float32),
                   jax.ShapeDtypeStruct((8, PH), jnp.float32)],
    )(u4, a0, a1, ba4, wbd, bb4)


def _bn_mm(h4, s, s2, g, be, summ, wbd):
    return pl.pallas_call(
        _bn_mm_body,
        grid=(GRID,),
        in_specs=[_row_spec(PH), _full_spec((8, PH)), _full_spec((8, PH)),
                  _full_spec((1, H)), _full_spec((1, H)),
                  _full_spec((PH, H)), _full_spec((PH, PH))],
        out_specs=_row_spec(PH),
        out_shape=jax.ShapeDtypeStruct((PN, PH), jnp.float32),
    )(h4, s, s2, g.reshape(1, H), be.reshape(1, H), summ, wbd)


def _head(h4, s, s2, g, be, summ, wf1d, bf1_4, wf2d, bf2_4):
    return pl.pallas_call(
        _head_body,
        grid=(GRID,),
        in_specs=[_row_spec(PH), _full_spec((8, PH)), _full_spec((8, PH)),
                  _full_spec((1, H)), _full_spec((1, H)),
                  _full_spec((PH, H)), _full_spec((PH, PH)),
                  _full_spec((1, PH)), _full_spec((PH, PK * C)),
                  _full_spec((1, PK * C))],
        out_specs=_row_spec(PK * C),
        out_shape=jax.ShapeDtypeStruct((PN, PK * C), jnp.float32),
    )(h4, s, s2, g.reshape(1, H), be.reshape(1, H), summ,
      wf1d, bf1_4, wf2d, bf2_4)


def kernel(x, edge_index, W1a, b1a, W1b, b1b, g1, be1,
           W2a, b2a, W2b, b2b, g2, be2, Wf1, bf1, Wf2, bf2):
    eidx = edge_index.reshape(2, NW, CHUNKS, EB)
    zeros = jnp.zeros((NP, H), jnp.float32)
    summ = jnp.tile(jnp.eye(H, dtype=jnp.float32), (PK, 1))

    w1ad = _blockdiag(W1a)
    w1bd = _blockdiag(W1b)
    w2ad = _blockdiag(W2a)
    w2bd = _blockdiag(W2b)
    wf1d = _blockdiag(Wf1)
    wf2d = _blockdiag(Wf2)
    b1a4 = jnp.tile(b1a, PK).reshape(1, PH)
    b1b4 = jnp.tile(b1b, PK).reshape(1, PH)
    b2a4 = jnp.tile(b2a, PK).reshape(1, PH)
    b2b4 = jnp.tile(b2b, PK).reshape(1, PH)
    bf14 = jnp.tile(bf1, PK).reshape(1, PH)
    bf24 = jnp.tile(bf2, PK).reshape(1, PK * C)

    u4 = _mm(x.reshape(PN, PK * D), w1ad)
    agg0, agg1 = _sc_agg(u4.reshape(N, H), eidx, zeros)
    h1, s1, s1q = _conv_post(u4, agg0.reshape(PNP, PH), agg1.reshape(PNP, PH),
                             b1a4, w1bd, b1b4)
    v2 = _bn_mm(h1, s1, s1q, g1, be1, summ, w2ad)
    agg0b, agg1b = _sc_agg(v2.reshape(N, H), eidx, zeros)
    h2, s2, s2q = _conv_post(v2, agg0b.reshape(PNP, PH),
                             agg1b.reshape(PNP, PH), b2a4, w2bd, b2b4)
    out4 = _head(h2, s2, s2q, g2, be2, summ, wf1d, bf14, wf2d, bf24)
    return out4.reshape(N, C)


# speedup vs baseline: 2.8969x; 1.0263x over previous
import functools

import jax
import jax.numpy as jnp
from jax import lax
from jax.experimental import pallas as pl
from jax.experimental.pallas import tpu as pltpu
from jax.experimental.pallas import tpu_sc as plsc

N = 10000
E = 320000
D = 128
H = 32
C = 40

NC = 2
NS = 16
NW = NC * NS
EB = 80
EPW = E // NW
CHUNKS = EPW // EB
NB = 5
HGROUPS = (CHUNKS // NB - 1) // 2
NP = 10112
RPT = NP // NS

PK = 4
PN = N // PK
PNP = NP // PK
PH = PK * H
PR = 512
GRID = -(-PN // PR)

_PREC = jax.lax.Precision.HIGHEST


def _sc_agg_body(u_hbm, eidx_hbm, zeros_hbm, out0_hbm, out1_hbm,
                 sidx_v, didx_v, rows_v, stage_v, acc_sh, gsems, ssems):
    c = lax.axis_index("c")
    s = lax.axis_index("s")
    wid = c * NS + s

    pltpu.sync_copy(zeros_hbm.at[pl.ds(s * RPT, RPT)], stage_v)
    pltpu.sync_copy(stage_v, acc_sh.at[pl.ds(s * RPT, RPT)])

    pltpu.sync_copy(eidx_hbm.at[0, wid], sidx_v)
    pltpu.sync_copy(eidx_hbm.at[1, wid], didx_v)
    plsc.subcore_barrier()

    def _gather(j, bank, b):
        pltpu.async_copy(u_hbm.at[sidx_v.at[j]], rows_v.at[bank, b],
                         gsems.at[bank, b])

    def _wait_gather(j, bank, b):
        pltpu.make_async_copy(u_hbm.at[sidx_v.at[j]], rows_v.at[bank, b],
                              gsems.at[bank, b]).wait()

    def _scatter(j, bank, b):
        pltpu.async_copy(rows_v.at[bank, b], acc_sh.at[didx_v.at[j]],
                         ssems.at[bank, b], add=True)

    def _wait_scatter(j, bank, b):
        pltpu.make_async_copy(rows_v.at[bank, b], acc_sh.at[didx_v.at[j]],
                              ssems.at[bank, b]).wait()

    for b in range(NB):
        _gather(b, 0, b)
    for b in range(NB):
        _gather(NB + b, 1, b)

    def body(t, carry):
        ja = 2 * NB * t
        jb = ja + NB
        for b in range(NB):
            _wait_gather(ja + b, 0, b)
            _scatter(ja + b, 0, b)
        for b in range(NB):
            _wait_scatter(ja + b, 0, b)
            _gather(ja + 2 * NB + b, 0, b)
        for b in range(NB):
            _wait_gather(jb + b, 1, b)
            _scatter(jb + b, 1, b)
        for b in range(NB):
            _wait_scatter(jb + b, 1, b)

            @pl.when(t + 1 < HGROUPS)
            def _():
                _gather(jb + 2 * NB + b, 1, b)

        return carry

    lax.fori_loop(0, HGROUPS, body, 0)

    jt = 2 * NB * HGROUPS
    for b in range(NB):
        _wait_gather(jt + b, 0, b)
        _scatter(jt + b, 0, b)
    for b in range(NB):
        _wait_scatter(jt + b, 0, b)
    plsc.subcore_barrier()

    pltpu.sync_copy(acc_sh.at[pl.ds(s * RPT, RPT)], stage_v)

    @pl.when(c == 0)
    def _():
        pltpu.sync_copy(stage_v, out0_hbm.at[pl.ds(s * RPT, RPT)])

    @pl.when(c == 1)
    def _():
        pltpu.sync_copy(stage_v, out1_hbm.at[pl.ds(s * RPT, RPT)])


_sc_agg = functools.partial(
    pl.kernel,
    out_type=[jax.ShapeDtypeStruct((NP, H), jnp.float32),
              jax.ShapeDtypeStruct((NP, H), jnp.float32)],
    mesh=plsc.VectorSubcoreMesh(core_axis_name="c", subcore_axis_name="s",
                                num_cores=NC, num_subcores=NS),
    scratch_types=[
        pltpu.VMEM((CHUNKS, EB), jnp.int32),
        pltpu.VMEM((CHUNKS, EB), jnp.int32),
        pltpu.VMEM((2, NB, EB, H), jnp.float32),
        pltpu.VMEM((RPT, H), jnp.float32),
        pltpu.VMEM_SHARED((NP, H), jnp.float32),
        pltpu.SemaphoreType.DMA((2, NB)),
        pltpu.SemaphoreType.DMA((2, NB)),
    ],
    compiler_params=pltpu.CompilerParams(use_tc_tiling_on_sc=False),
)(_sc_agg_body)


def _mm_body(x_ref, w_ref, o_ref):
    o_ref[...] = jnp.dot(x_ref[...], w_ref[...],
                         preferred_element_type=jnp.float32, precision=_PREC)


def _stats(h, i, s_ref, s2_ref):
    @pl.when(i == 0)
    def _():
        s_ref[...] = jnp.zeros_like(s_ref)
        s2_ref[...] = jnp.zeros_like(s2_ref)

    ri = jax.lax.broadcasted_iota(jnp.int32, (PR, 1), 0) + i * PR
    hm = jnp.where(ri < PN, h, 0.0)
    hr = hm.reshape(PR // 8, 8, PH)
    s_ref[...] += jnp.sum(hr, axis=0)
    s2_ref[...] += jnp.sum(hr * hr, axis=0)


def _conv_bn_body(u_ref, a0_ref, a1_ref, ba_ref, wbd_ref, bb_ref,
                  g_ref, be_ref, summ_ref, w2d_ref, o_ref,
                  hscr, sscr, s2scr):
    i = pl.program_id(0)

    @pl.when(i < GRID)
    def _():
        z = jnp.maximum(
            u_ref[...] + a0_ref[...] + a1_ref[...] + ba_ref[...], 0.0)
        h = jnp.dot(z, wbd_ref[...], preferred_element_type=jnp.float32,
                    precision=_PREC) + bb_ref[...]
        hscr[pl.ds(i * PR, PR), :] = h
        _stats(h, i, sscr, s2scr)

    @pl.when(i >= GRID)
    def _():
        scale128, shift128 = _bn_scale_shift(sscr, s2scr, g_ref, be_ref,
                                             summ_ref)
        h = hscr[pl.ds((i - GRID) * PR, PR), :]
        hn = h * scale128 + shift128
        o_ref[...] = jnp.dot(hn, w2d_ref[...],
                             preferred_element_type=jnp.float32,
                             precision=_PREC)


def _bn_scale_shift(s_ref, s2_ref, g_ref, be_ref, summ_ref):
    summ = summ_ref[...]
    s32 = jnp.dot(jnp.sum(s_ref[...], axis=0, keepdims=True), summ,
                  preferred_element_type=jnp.float32, precision=_PREC)
    q32 = jnp.dot(jnp.sum(s2_ref[...], axis=0, keepdims=True), summ,
                  preferred_element_type=jnp.float32, precision=_PREC)
    m = s32 / N
    var = q32 / N - m * m
    scale = g_ref[...] * jax.lax.rsqrt(var + 1e-5)
    shift = be_ref[...] - m * scale
    rep = summ.T
    scale128 = jnp.dot(scale, rep, preferred_element_type=jnp.float32,
                       precision=_PREC)
    shift128 = jnp.dot(shift, rep, preferred_element_type=jnp.float32,
                       precision=_PREC)
    return scale128, shift128


def _conv_head_body(u_ref, a0_ref, a1_ref, ba_ref, wbd_ref, bb_ref,
                    g_ref, be_ref, summ_ref, wf1d_ref, bf1_ref,
                    wf2d_ref, bf2_ref, o_ref, hscr, sscr, s2scr):
    i = pl.program_id(0)

    @pl.when(i < GRID)
    def _():
        z = jnp.maximum(
            u_ref[...] + a0_ref[...] + a1_ref[...] + ba_ref[...], 0.0)
        h = jnp.dot(z, wbd_ref[...], preferred_element_type=jnp.float32,
                    precision=_PREC) + bb_ref[...]
        hscr[pl.ds(i * PR, PR), :] = h
        _stats(h, i, sscr, s2scr)

    @pl.when(i >= GRID)
    def _():
        scale128, shift128 = _bn_scale_shift(sscr, s2scr, g_ref, be_ref,
                                             summ_ref)
        h = hscr[pl.ds((i - GRID) * PR, PR), :]
        hn = h * scale128 + shift128
        f = jnp.maximum(
            jnp.dot(hn, wf1d_ref[...], preferred_element_type=jnp.float32,
                    precision=_PREC) + bf1_ref[...], 0.0)
        o_ref[...] = jnp.dot(f, wf2d_ref[...],
                             preferred_element_type=jnp.float32,
                             precision=_PREC) + bf2_ref[...]


def _row_spec(width):
    return pl.BlockSpec((PR, width), lambda i: (i, 0))


def _full_spec(shape):
    return pl.BlockSpec(shape, lambda i: tuple(0 for _ in shape))


def _blockdiag(w):
    h, wdt = w.shape
    z = jnp.zeros((h, wdt), jnp.float32)
    rows = [jnp.concatenate([w if j == k else z for j in range(PK)], axis=1)
            for k in range(PK)]
    return jnp.concatenate(rows, axis=0)


def _mm(x4, wbd):
    return pl.pallas_call(
        _mm_body,
        grid=(GRID,),
        in_specs=[_row_spec(PK * D), _full_spec(wbd.shape)],
        out_specs=_row_spec(PH),
        out_shape=jax.ShapeDtypeStruct((PN, PH), jnp.float32),
    )(x4, wbd)


def _phase_row_spec(width):
    return pl.BlockSpec((PR, width), lambda i: (i % GRID, 0))


def _phase2_row_spec(width):
    return pl.BlockSpec((PR, width), lambda i: (i % GRID, 0))


_CONV_SCRATCH = [
    pltpu.VMEM((GRID * PR, PH), jnp.float32),
    pltpu.VMEM((8, PH), jnp.float32),
    pltpu.VMEM((8, PH), jnp.float32),
]


def _conv_bn(u4, a0, a1, ba4, wbd, bb4, g, be, summ, w2d):
    return pl.pallas_call(
        _conv_bn_body,
        grid=(2 * GRID,),
        in_specs=[_phase_row_spec(PH), _phase_row_spec(PH),
                  _phase_row_spec(PH),
                  _full_spec((1, PH)), _full_spec((PH, PH)),
                  _full_spec((1, PH)), _full_spec((1, H)),
                  _full_spec((1, H)), _full_spec((PH, H)),
                  _full_spec((PH, PH))],
        out_specs=_phase2_row_spec(PH),
        out_shape=jax.ShapeDtypeStruct((PN, PH), jnp.float32),
        scratch_shapes=_CONV_SCRATCH,
    )(u4, a0, a1, ba4, wbd, bb4, g.reshape(1, H), be.reshape(1, H),
      summ, w2d)


def _conv_head(u4, a0, a1, ba4, wbd, bb4, g, be, summ,
               wf1d, bf1_4, wf2d, bf2_4):
    return pl.pallas_call(
        _conv_head_body,
        grid=(2 * GRID,),
        in_specs=[_phase_row_spec(PH), _phase_row_spec(PH),
                  _phase_row_spec(PH),
                  _full_spec((1, PH)), _full_spec((PH, PH)),
                  _full_spec((1, PH)), _full_spec((1, H)),
                  _full_spec((1, H)), _full_spec((PH, H)),
                  _full_spec((PH, PH)), _full_spec((1, PH)),
                  _full_spec((PH, PK * C)), _full_spec((1, PK * C))],
        out_specs=_phase2_row_spec(PK * C),
        out_shape=jax.ShapeDtypeStruct((PN, PK * C), jnp.float32),
        scratch_shapes=_CONV_SCRATCH,
    )(u4, a0, a1, ba4, wbd, bb4, g.reshape(1, H), be.reshape(1, H),
      summ, wf1d, bf1_4, wf2d, bf2_4)


def kernel(x, edge_index, W1a, b1a, W1b, b1b, g1, be1,
           W2a, b2a, W2b, b2b, g2, be2, Wf1, bf1, Wf2, bf2):
    eidx = edge_index.reshape(2, NW, CHUNKS, EB)
    zeros = jnp.zeros((NP, H), jnp.float32)
    summ = jnp.tile(jnp.eye(H, dtype=jnp.float32), (PK, 1))

    w1ad = _blockdiag(W1a)
    w1bd = _blockdiag(W1b)
    w2ad = _blockdiag(W2a)
    w2bd = _blockdiag(W2b)
    wf1d = _blockdiag(Wf1)
    wf2d = _blockdiag(Wf2)
    b1a4 = jnp.tile(b1a, PK).reshape(1, PH)
    b1b4 = jnp.tile(b1b, PK).reshape(1, PH)
    b2a4 = jnp.tile(b2a, PK).reshape(1, PH)
    b2b4 = jnp.tile(b2b, PK).reshape(1, PH)
    bf14 = jnp.tile(bf1, PK).reshape(1, PH)
    bf24 = jnp.tile(bf2, PK).reshape(1, PK * C)

    u4 = _mm(x.reshape(PN, PK * D), w1ad)
    agg0, agg1 = _sc_agg(u4.reshape(N, H), eidx, zeros)
    v2 = _conv_bn(u4, agg0.reshape(PNP, PH), agg1.reshape(PNP, PH),
                  b1a4, w1bd, b1b4, g1, be1, summ, w2ad)
    agg0b, agg1b = _sc_agg(v2.reshape(N, H), eidx, zeros)
    out4 = _conv_head(v2, agg0b.reshape(PNP, PH), agg1b.reshape(PNP, PH),
                      b2a4, w2bd, b2b4, g2, be2, summ,
                      wf1d, bf14, wf2d, bf24)
    return out4.reshape(N, C)


# speedup vs baseline: 2.9413x; 1.0154x over previous
import functools

import jax
import jax.numpy as jnp
from jax import lax
from jax.experimental import pallas as pl
from jax.experimental.pallas import tpu as pltpu
from jax.experimental.pallas import tpu_sc as plsc

N = 10000
E = 320000
D = 128
H = 32
C = 40

NC = 2
NS = 16
NW = NC * NS
EB = 80
EPW = E // NW
CHUNKS = EPW // EB
NB = 5
HGROUPS = (CHUNKS // NB - 1) // 2
NP = 10112
RPT = NP // NS

PK = 4
PN = N // PK
PNP = NP // PK
PH = PK * H
PR = 512
GRID = -(-PN // PR)

_PREC = jax.lax.Precision.HIGHEST


def _sc_agg_body(u_hbm, eidx_hbm, zeros_hbm, out0_hbm, out1_hbm,
                 sidx_v, didx_v, rows_v, stage_v, acc_sh, gsems, ssems):
    c = lax.axis_index("c")
    s = lax.axis_index("s")
    wid = c * NS + s

    pltpu.sync_copy(zeros_hbm.at[pl.ds(s * RPT, RPT)], stage_v)
    pltpu.sync_copy(stage_v, acc_sh.at[pl.ds(s * RPT, RPT)])

    pltpu.sync_copy(eidx_hbm.at[0, wid], sidx_v)
    pltpu.sync_copy(eidx_hbm.at[1, wid], didx_v)
    plsc.subcore_barrier()

    def _gather(j, bank, b):
        pltpu.async_copy(u_hbm.at[sidx_v.at[j]], rows_v.at[bank, b],
                         gsems.at[bank, b])

    def _wait_gather(j, bank, b):
        pltpu.make_async_copy(u_hbm.at[sidx_v.at[j]], rows_v.at[bank, b],
                              gsems.at[bank, b]).wait()

    def _scatter(j, bank, b):
        pltpu.async_copy(rows_v.at[bank, b], acc_sh.at[didx_v.at[j]],
                         ssems.at[bank, b], add=True)

    def _wait_scatter(j, bank, b):
        pltpu.make_async_copy(rows_v.at[bank, b], acc_sh.at[didx_v.at[j]],
                              ssems.at[bank, b]).wait()

    for b in range(NB):
        _gather(b, 0, b)
    for b in range(NB):
        _gather(NB + b, 1, b)

    def body(t, carry):
        ja = 2 * NB * t
        jb = ja + NB
        for b in range(NB):
            _wait_gather(ja + b, 0, b)
            _scatter(ja + b, 0, b)
        for b in range(NB):
            _wait_scatter(ja + b, 0, b)
            _gather(ja + 2 * NB + b, 0, b)
        for b in range(NB):
            _wait_gather(jb + b, 1, b)
            _scatter(jb + b, 1, b)
        for b in range(NB):
            _wait_scatter(jb + b, 1, b)

            @pl.when(t + 1 < HGROUPS)
            def _():
                _gather(jb + 2 * NB + b, 1, b)

        return carry

    lax.fori_loop(0, HGROUPS, body, 0)

    jt = 2 * NB * HGROUPS
    for b in range(NB):
        _wait_gather(jt + b, 0, b)
        _scatter(jt + b, 0, b)
    for b in range(NB):
        _wait_scatter(jt + b, 0, b)
    plsc.subcore_barrier()

    pltpu.sync_copy(acc_sh.at[pl.ds(s * RPT, RPT)], stage_v)

    @pl.when(c == 0)
    def _():
        pltpu.sync_copy(stage_v, out0_hbm.at[pl.ds(s * RPT, RPT)])

    @pl.when(c == 1)
    def _():
        pltpu.sync_copy(stage_v, out1_hbm.at[pl.ds(s * RPT, RPT)])


_sc_agg = functools.partial(
    pl.kernel,
    out_type=[jax.ShapeDtypeStruct((NP, H), jnp.float32),
              jax.ShapeDtypeStruct((NP, H), jnp.float32)],
    mesh=plsc.VectorSubcoreMesh(core_axis_name="c", subcore_axis_name="s",
                                num_cores=NC, num_subcores=NS),
    scratch_types=[
        pltpu.VMEM((CHUNKS, EB), jnp.int32),
        pltpu.VMEM((CHUNKS, EB), jnp.int32),
        pltpu.VMEM((2, NB, EB, H), jnp.float32),
        pltpu.VMEM((RPT, H), jnp.float32),
        pltpu.VMEM_SHARED((NP, H), jnp.float32),
        pltpu.SemaphoreType.DMA((2, NB)),
        pltpu.SemaphoreType.DMA((2, NB)),
    ],
    compiler_params=pltpu.CompilerParams(use_tc_tiling_on_sc=False),
)(_sc_agg_body)


def _mm_body(x_ref, w_ref, o_ref):
    o_ref[...] = jnp.dot(x_ref[...], w_ref[...],
                         preferred_element_type=jnp.float32, precision=_PREC)


def _stats(h, i, s_ref, s2_ref):
    @pl.when(i == 0)
    def _():
        s_ref[...] = jnp.zeros_like(s_ref)
        s2_ref[...] = jnp.zeros_like(s2_ref)

    ri = jax.lax.broadcasted_iota(jnp.int32, (PR, 1), 0) + i * PR
    hm = jnp.where(ri < PN, h, 0.0)
    hr = hm.reshape(PR // 8, 8, PH)
    s_ref[...] += jnp.sum(hr, axis=0)
    s2_ref[...] += jnp.sum(hr * hr, axis=0)


def _conv_bn_body(u_ref, a0_ref, a1_ref, ba_ref, wbd_ref, bb_ref,
                  g_ref, be_ref, summ_ref, w2d_ref, o_ref,
                  hscr, sscr, s2scr):
    i = pl.program_id(0)

    @pl.when(i < GRID)
    def _():
        z = jnp.maximum(
            u_ref[...] + a0_ref[...] + a1_ref[...] + ba_ref[...], 0.0)
        h = jnp.dot(z, wbd_ref[...], preferred_element_type=jnp.float32,
                    precision=_PREC) + bb_ref[...]
        hscr[pl.ds(i * PR, PR), :] = h
        _stats(h, i, sscr, s2scr)

    @pl.when(i >= GRID)
    def _():
        scale128, shift128 = _bn_scale_shift(sscr, s2scr, g_ref, be_ref,
                                             summ_ref)
        h = hscr[pl.ds((i - GRID) * PR, PR), :]
        hn = h * scale128 + shift128
        o_ref[...] = jnp.dot(hn, w2d_ref[...],
                             preferred_element_type=jnp.float32,
                             precision=_PREC)


def _bn_scale_shift(s_ref, s2_ref, g_ref, be_ref, summ_ref):
    summ = summ_ref[...]
    s32 = jnp.dot(jnp.sum(s_ref[...], axis=0, keepdims=True), summ,
                  preferred_element_type=jnp.float32, precision=_PREC)
    q32 = jnp.dot(jnp.sum(s2_ref[...], axis=0, keepdims=True), summ,
                  preferred_element_type=jnp.float32, precision=_PREC)
    m = s32 / N
    var = q32 / N - m * m
    scale = g_ref[...] * jax.lax.rsqrt(var + 1e-5)
    shift = be_ref[...] - m * scale
    rep = summ.T
    scale128 = jnp.dot(scale, rep, preferred_element_type=jnp.float32,
                       precision=_PREC)
    shift128 = jnp.dot(shift, rep, preferred_element_type=jnp.float32,
                       precision=_PREC)
    return scale128, shift128


def _conv_head_body(u_ref, a0_ref, a1_ref, ba_ref, wbd_ref, bb_ref,
                    g_ref, be_ref, summ_ref, wf1d_ref, bf1_ref,
                    wf2d_ref, bf2_ref, o_ref, hscr, sscr, s2scr):
    i = pl.program_id(0)

    @pl.when(i < GRID)
    def _():
        z = jnp.maximum(
            u_ref[...] + a0_ref[...] + a1_ref[...] + ba_ref[...], 0.0)
        h = jnp.dot(z, wbd_ref[...], preferred_element_type=jnp.float32,
                    precision=_PREC) + bb_ref[...]
        hscr[pl.ds(i * PR, PR), :] = h
        _stats(h, i, sscr, s2scr)

    @pl.when(i >= GRID)
    def _():
        scale128, shift128 = _bn_scale_shift(sscr, s2scr, g_ref, be_ref,
                                             summ_ref)
        h = hscr[pl.ds((i - GRID) * PR, PR), :]
        hn = h * scale128 + shift128
        f = jnp.maximum(
            jnp.dot(hn, wf1d_ref[...], preferred_element_type=jnp.float32,
                    precision=_PREC) + bf1_ref[...], 0.0)
        o_ref[...] = jnp.dot(f, wf2d_ref[...],
                             preferred_element_type=jnp.float32,
                             precision=_PREC) + bf2_ref[...]


def _row_spec(width):
    return pl.BlockSpec((PR, width), lambda i: (i, 0))


def _full_spec(shape):
    return pl.BlockSpec(shape, lambda i: tuple(0 for _ in shape))


def _blockdiag(w):
    h, wdt = w.shape
    z = jnp.zeros((h, wdt), jnp.float32)
    rows = [jnp.concatenate([w if j == k else z for j in range(PK)], axis=1)
            for k in range(PK)]
    return jnp.concatenate(rows, axis=0)


def _mm(x4, wbd):
    return pl.pallas_call(
        _mm_body,
        grid=(GRID,),
        in_specs=[_row_spec(PK * D), _full_spec(wbd.shape)],
        out_specs=_row_spec(PH),
        out_shape=jax.ShapeDtypeStruct((PN, PH), jnp.float32),
    )(x4, wbd)


def _phase_row_spec(width):
    return pl.BlockSpec((PR, width),
                        lambda i: (jnp.minimum(i, GRID - 1), 0))


def _phase2_row_spec(width):
    return pl.BlockSpec((PR, width),
                        lambda i: (jnp.maximum(i - GRID, 0), 0))


_CONV_SCRATCH = [
    pltpu.VMEM((GRID * PR, PH), jnp.float32),
    pltpu.VMEM((8, PH), jnp.float32),
    pltpu.VMEM((8, PH), jnp.float32),
]


def _conv_bn(u4, a0, a1, ba4, wbd, bb4, g, be, summ, w2d):
    return pl.pallas_call(
        _conv_bn_body,
        grid=(2 * GRID,),
        in_specs=[_phase_row_spec(PH), _phase_row_spec(PH),
                  _phase_row_spec(PH),
                  _full_spec((1, PH)), _full_spec((PH, PH)),
                  _full_spec((1, PH)), _full_spec((1, H)),
                  _full_spec((1, H)), _full_spec((PH, H)),
                  _full_spec((PH, PH))],
        out_specs=_phase2_row_spec(PH),
        out_shape=jax.ShapeDtypeStruct((PN, PH), jnp.float32),
        scratch_shapes=_CONV_SCRATCH,
    )(u4, a0, a1, ba4, wbd, bb4, g.reshape(1, H), be.reshape(1, H),
      summ, w2d)


def _conv_head(u4, a0, a1, ba4, wbd, bb4, g, be, summ,
               wf1d, bf1_4, wf2d, bf2_4):
    return pl.pallas_call(
        _conv_head_body,
        grid=(2 * GRID,),
        in_specs=[_phase_row_spec(PH), _phase_row_spec(PH),
                  _phase_row_spec(PH),
                  _full_spec((1, PH)), _full_spec((PH, PH)),
                  _full_spec((1, PH)), _full_spec((1, H)),
                  _full_spec((1, H)), _full_spec((PH, H)),
                  _full_spec((PH, PH)), _full_spec((1, PH)),
                  _full_spec((PH, PK * C)), _full_spec((1, PK * C))],
        out_specs=_phase2_row_spec(PK * C),
        out_shape=jax.ShapeDtypeStruct((PN, PK * C), jnp.float32),
        scratch_shapes=_CONV_SCRATCH,
    )(u4, a0, a1, ba4, wbd, bb4, g.reshape(1, H), be.reshape(1, H),
      summ, wf1d, bf1_4, wf2d, bf2_4)


def kernel(x, edge_index, W1a, b1a, W1b, b1b, g1, be1,
           W2a, b2a, W2b, b2b, g2, be2, Wf1, bf1, Wf2, bf2):
    eidx = edge_index.reshape(2, NW, CHUNKS, EB)
    zeros = jnp.zeros((NP, H), jnp.float32)
    summ = jnp.tile(jnp.eye(H, dtype=jnp.float32), (PK, 1))

    w1ad = _blockdiag(W1a)
    w1bd = _blockdiag(W1b)
    w2ad = _blockdiag(W2a)
    w2bd = _blockdiag(W2b)
    wf1d = _blockdiag(Wf1)
    wf2d = _blockdiag(Wf2)
    b1a4 = jnp.tile(b1a, PK).reshape(1, PH)
    b1b4 = jnp.tile(b1b, PK).reshape(1, PH)
    b2a4 = jnp.tile(b2a, PK).reshape(1, PH)
    b2b4 = jnp.tile(b2b, PK).reshape(1, PH)
    bf14 = jnp.tile(bf1, PK).reshape(1, PH)
    bf24 = jnp.tile(bf2, PK).reshape(1, PK * C)

    u4 = _mm(x.reshape(PN, PK * D), w1ad)
    agg0, agg1 = _sc_agg(u4.reshape(N, H), eidx, zeros)
    v2 = _conv_bn(u4, agg0.reshape(PNP, PH), agg1.reshape(PNP, PH),
                  b1a4, w1bd, b1b4, g1, be1, summ, w2ad)
    agg0b, agg1b = _sc_agg(v2.reshape(N, H), eidx, zeros)
    out4 = _conv_head(v2, agg0b.reshape(PNP, PH), agg1b.reshape(PNP, PH),
                      b2a4, w2bd, b2b4, g2, be2, summ,
                      wf1d, bf14, wf2d, bf24)
    return out4.reshape(N, C)
